# baseline (speedup 1.0000x reference)
import functools

import jax
import jax.numpy as jnp
from jax import lax
from jax.experimental import pallas as pl
from jax.experimental.pallas import tpu as pltpu
from jax.experimental.pallas import tpu_sc as plsc

EMB = 128
NC = 2
NS = 16
NW = NC * NS
CHUNK = 128


DEPTH = 4


def _make_gather(n_rows: int):
    assert n_rows % (NW * CHUNK) == 0
    rows_per_w = n_rows // NW
    n_chunks = rows_per_w // CHUNK
    assert n_chunks % DEPTH == 2 or n_chunks % DEPTH == 0
    tail = n_chunks % DEPTH
    mesh = plsc.VectorSubcoreMesh(core_axis_name="c", subcore_axis_name="s")

    @functools.partial(
        pl.kernel,
        mesh=mesh,
        out_type=jax.ShapeDtypeStruct((n_rows, EMB), jnp.float32),
        scratch_types=[
            pltpu.VMEM((n_chunks, CHUNK), jnp.int32),
        ]
        + [pltpu.VMEM((CHUNK, EMB), jnp.float32) for _ in range(DEPTH)]
        + [pltpu.SemaphoreType.DMA for _ in range(2 * DEPTH)],
    )
    def gather(table_hbm, idx_hbm, out_hbm, idx_v, *rest):
        bufs = rest[:DEPTH]
        gsems = rest[DEPTH : 2 * DEPTH]
        osems = rest[2 * DEPTH :]
        wid = lax.axis_index("s") * NC + lax.axis_index("c")
        base = wid * rows_per_w
        pltpu.sync_copy(idx_hbm.at[wid], idx_v)

        def start_gather(c, b):
            pltpu.async_copy(table_hbm.at[idx_v.at[c]], bufs[b], gsems[b])

        def wait_gather(c, b):
            pltpu.make_async_copy(table_hbm.at[idx_v.at[c]], bufs[b], gsems[b]).wait()

        def out_slice(c):
            return out_hbm.at[pl.ds(base + c * CHUNK, CHUNK)]

        def start_out(c, b):
            pltpu.async_copy(bufs[b], out_slice(c), osems[b])

        def wait_out(c, b):
            pltpu.make_async_copy(bufs[b], out_slice(c), osems[b]).wait()

        for b in range(DEPTH):
            start_gather(b, b)

        def body(i, _):
            for b in range(DEPTH):
                c = i * DEPTH + b
                wait_gather(c, b)
                start_out(c, b)

                @pl.when(c + DEPTH < n_chunks)
                def _():
                    wait_out(c, b)
                    start_gather(c + DEPTH, b)

            return 0

        lax.fori_loop(0, n_chunks // DEPTH, body, 0)

        for t in range(tail):
            c = (n_chunks // DEPTH) * DEPTH + t
            wait_gather(c, t)
            start_out(c, t)

        for t in range(DEPTH):
            c = n_chunks - DEPTH + t
            wait_out(c, c % DEPTH)

    return gather


@jax.jit
def kernel(word_sequences, table):
    n_seq, seq_len = word_sequences.shape
    pad = (-seq_len) % 8
    seq_pad = seq_len + pad
    n_rows = n_seq * seq_pad
    idx = jnp.pad(word_sequences.astype(jnp.int32), ((0, 0), (0, pad)))
    idx = idx.reshape(NW, n_rows // (NW * CHUNK), CHUNK)
    out = _make_gather(n_rows)(table, idx)
    return out.reshape(n_seq, seq_pad, EMB)[:, :seq_len, :]


# speedup vs baseline: 7.8163x; 7.8163x over previous
import functools

import jax
import jax.numpy as jnp
from jax import lax
from jax.experimental import pallas as pl
from jax.experimental.pallas import tpu as pltpu
from jax.experimental.pallas import tpu_sc as plsc

EMB = 128
NC = 2
NS = 16
NW = NC * NS
DEPTH = 4


def _make_gather(n_seq: int, seq_len: int):
    assert n_seq % NW == 0
    seq_per_w = n_seq // NW
    assert seq_per_w % DEPTH == 0
    row_pad = 128
    mesh = plsc.VectorSubcoreMesh(core_axis_name="c", subcore_axis_name="s")

    @functools.partial(
        pl.kernel,
        mesh=mesh,
        out_type=jax.ShapeDtypeStruct((n_seq, seq_len, EMB), jnp.float32),
        scratch_types=[
            pltpu.VMEM((seq_per_w, row_pad), jnp.int32),
        ]
        + [pltpu.VMEM((seq_len, EMB), jnp.float32) for _ in range(DEPTH)]
        + [pltpu.SemaphoreType.DMA for _ in range(2 * DEPTH)],
    )
    def gather(table_hbm, idx_hbm, out_hbm, idx_v, *rest):
        bufs = rest[:DEPTH]
        gsems = rest[DEPTH : 2 * DEPTH]
        osems = rest[2 * DEPTH :]
        wid = lax.axis_index("s") * NC + lax.axis_index("c")
        base = wid * seq_per_w
        pltpu.sync_copy(idx_hbm.at[wid], idx_v)

        def idx_row(j):
            return idx_v.at[j, pl.ds(0, seq_len)]

        def start_gather(j, b):
            pltpu.async_copy(table_hbm.at[idx_row(j)], bufs[b], gsems[b])

        def wait_gather(j, b):
            pltpu.make_async_copy(table_hbm.at[idx_row(j)], bufs[b], gsems[b]).wait()

        def start_out(j, b):
            pltpu.async_copy(bufs[b], out_hbm.at[base + j], osems[b])

        def wait_out(j, b):
            pltpu.make_async_copy(bufs[b], out_hbm.at[base + j], osems[b]).wait()

        for b in range(DEPTH):
            start_gather(b, b)

        def body(i, _):
            for b in range(DEPTH):
                j = i * DEPTH + b
                wait_gather(j, b)
                start_out(j, b)

                @pl.when(j + DEPTH < seq_per_w)
                def _():
                    wait_out(j, b)
                    start_gather(j + DEPTH, b)

            return 0

        lax.fori_loop(0, seq_per_w // DEPTH, body, 0)

        for t in range(DEPTH):
            j = seq_per_w - DEPTH + t
            wait_out(j, j % DEPTH)

    return gather


@jax.jit
def kernel(word_sequences, table):
    n_seq, seq_len = word_sequences.shape
    idx = jnp.pad(word_sequences.astype(jnp.int32), ((0, 0), (0, 128 - seq_len)))
    idx = idx.reshape(NW, n_seq // NW, 128)
    return _make_gather(n_seq, seq_len)(table, idx)


# speedup vs baseline: 13.7627x; 1.7608x over previous
import functools

import jax
import jax.numpy as jnp
from jax import lax
from jax.experimental import pallas as pl
from jax.experimental.pallas import tpu as pltpu
from jax.experimental.pallas import tpu_sc as plsc

EMB = 128
NC = 2
NS = 16
NW = NC * NS
CHUNK = 128
DEPTH = 4


def _make_gather(n_rows: int):
    assert n_rows % (NW * CHUNK) == 0
    rows_per_w = n_rows // NW
    n_chunks = rows_per_w // CHUNK
    assert n_chunks % DEPTH == 2 or n_chunks % DEPTH == 0
    tail = n_chunks % DEPTH
    mesh = plsc.VectorSubcoreMesh(core_axis_name="c", subcore_axis_name="s")

    @functools.partial(
        pl.kernel,
        mesh=mesh,
        out_type=jax.ShapeDtypeStruct((n_rows, EMB), jnp.float32),
        scratch_types=[
            pltpu.VMEM((n_chunks, CHUNK), jnp.int32),
        ]
        + [pltpu.VMEM((CHUNK, EMB), jnp.float32) for _ in range(DEPTH)]
        + [pltpu.SemaphoreType.DMA for _ in range(2 * DEPTH)],
    )
    def gather(table_hbm, idx_hbm, out_hbm, idx_v, *rest):
        bufs = rest[:DEPTH]
        gsems = rest[DEPTH : 2 * DEPTH]
        osems = rest[2 * DEPTH :]
        wid = lax.axis_index("s") * NC + lax.axis_index("c")
        base = wid * rows_per_w
        pltpu.sync_copy(idx_hbm.at[wid], idx_v)

        def start_gather(c, b):
            pltpu.async_copy(table_hbm.at[idx_v.at[c]], bufs[b], gsems[b])

        def wait_gather(c, b):
            pltpu.make_async_copy(table_hbm.at[idx_v.at[c]], bufs[b], gsems[b]).wait()

        def out_slice(c):
            return out_hbm.at[pl.ds(base + c * CHUNK, CHUNK)]

        def start_out(c, b):
            pltpu.async_copy(bufs[b], out_slice(c), osems[b])

        def wait_out(c, b):
            pltpu.make_async_copy(bufs[b], out_slice(c), osems[b]).wait()

        for b in range(DEPTH):
            start_gather(b, b)

        def body(i, _):
            for b in range(DEPTH):
                c = i * DEPTH + b
                wait_gather(c, b)
                start_out(c, b)

                @pl.when(c + DEPTH < n_chunks)
                def _():
                    wait_out(c, b)
                    start_gather(c + DEPTH, b)

            return 0

        lax.fori_loop(0, n_chunks // DEPTH, body, 0)

        for t in range(tail):
            c = (n_chunks // DEPTH) * DEPTH + t
            wait_gather(c, t)
            start_out(c, t)

        for t in range(DEPTH):
            c = n_chunks - DEPTH + t
            wait_out(c, c % DEPTH)

    return gather


@jax.jit
def kernel(word_sequences, table):
    n_seq, seq_len = word_sequences.shape
    n_rows = n_seq * seq_len
    idx = word_sequences.astype(jnp.int32).T.reshape(
        NW, n_rows // (NW * CHUNK), CHUNK
    )
    out = _make_gather(n_rows)(table, idx)
    return out.reshape(seq_len, n_seq, EMB).transpose(1, 0, 2)


# speedup vs baseline: 13.8903x; 1.0093x over previous
import functools

import jax
import jax.numpy as jnp
from jax import lax
from jax.experimental import pallas as pl
from jax.experimental.pallas import tpu as pltpu
from jax.experimental.pallas import tpu_sc as plsc

EMB = 128
NC = 2
NS = 16
NW = NC * NS
CHUNK = 128
DEPTH = 6


def _make_gather(n_rows: int):
    assert n_rows % (NW * CHUNK) == 0
    rows_per_w = n_rows // NW
    n_chunks = rows_per_w // CHUNK
    assert n_chunks % DEPTH == 2 or n_chunks % DEPTH == 0
    tail = n_chunks % DEPTH
    mesh = plsc.VectorSubcoreMesh(core_axis_name="c", subcore_axis_name="s")

    @functools.partial(
        pl.kernel,
        mesh=mesh,
        out_type=jax.ShapeDtypeStruct((n_rows, EMB), jnp.float32),
        scratch_types=[
            pltpu.VMEM((n_chunks, CHUNK), jnp.int32),
        ]
        + [pltpu.VMEM((CHUNK, EMB), jnp.float32) for _ in range(DEPTH)]
        + [pltpu.SemaphoreType.DMA for _ in range(2 * DEPTH)],
    )
    def gather(table_hbm, idx_hbm, out_hbm, idx_v, *rest):
        bufs = rest[:DEPTH]
        gsems = rest[DEPTH : 2 * DEPTH]
        osems = rest[2 * DEPTH :]
        wid = lax.axis_index("s") * NC + lax.axis_index("c")
        base = wid * rows_per_w
        pltpu.sync_copy(idx_hbm.at[wid], idx_v)

        def start_gather(c, b):
            pltpu.async_copy(table_hbm.at[idx_v.at[c]], bufs[b], gsems[b])

        def wait_gather(c, b):
            pltpu.make_async_copy(table_hbm.at[idx_v.at[c]], bufs[b], gsems[b]).wait()

        def out_slice(c):
            return out_hbm.at[pl.ds(base + c * CHUNK, CHUNK)]

        def start_out(c, b):
            pltpu.async_copy(bufs[b], out_slice(c), osems[b])

        def wait_out(c, b):
            pltpu.make_async_copy(bufs[b], out_slice(c), osems[b]).wait()

        for b in range(DEPTH):
            start_gather(b, b)

        def body(i, _):
            for b in range(DEPTH):
                c = i * DEPTH + b
                wait_gather(c, b)
                start_out(c, b)

                @pl.when(c + DEPTH < n_chunks)
                def _():
                    wait_out(c, b)
                    start_gather(c + DEPTH, b)

            return 0

        lax.fori_loop(0, n_chunks // DEPTH, body, 0)

        for t in range(tail):
            c = (n_chunks // DEPTH) * DEPTH + t
            wait_gather(c, t)
            start_out(c, t)

        for t in range(DEPTH):
            c = n_chunks - DEPTH + t
            wait_out(c, c % DEPTH)

    return gather


@jax.jit
def kernel(word_sequences, table):
    n_seq, seq_len = word_sequences.shape
    n_rows = n_seq * seq_len
    idx = word_sequences.astype(jnp.int32).T.reshape(
        NW, n_rows // (NW * CHUNK), CHUNK
    )
    out = _make_gather(n_rows)(table, idx)
    return out.reshape(seq_len, n_seq, EMB).transpose(1, 0, 2)


# speedup vs baseline: 14.1369x; 1.0178x over previous
import functools

import jax
import jax.numpy as jnp
from jax import lax
from jax.experimental import pallas as pl
from jax.experimental.pallas import tpu as pltpu
from jax.experimental.pallas import tpu_sc as plsc

EMB = 128
NC = 2
NS = 16
NW = NC * NS
CHUNK = 128
DEPTH = 6


def _make_gather(n_seq: int, seq_len: int):
    assert n_seq % (NW * CHUNK) == 0 and n_seq // NW == CHUNK
    n_rows = n_seq * seq_len
    n_chunks = seq_len
    assert n_chunks % DEPTH == 2 or n_chunks % DEPTH == 0
    tail = n_chunks % DEPTH
    mesh = plsc.VectorSubcoreMesh(core_axis_name="c", subcore_axis_name="s")

    @functools.partial(
        pl.kernel,
        mesh=mesh,
        out_type=jax.ShapeDtypeStruct((n_rows, EMB), jnp.float32),
        scratch_types=[
            pltpu.VMEM((n_chunks, CHUNK), jnp.int32),
        ]
        + [pltpu.VMEM((CHUNK, EMB), jnp.float32) for _ in range(DEPTH)]
        + [pltpu.SemaphoreType.DMA for _ in range(2 * DEPTH)],
    )
    def gather(table_hbm, idx_hbm, out_hbm, idx_v, *rest):
        bufs = rest[:DEPTH]
        gsems = rest[DEPTH : 2 * DEPTH]
        osems = rest[2 * DEPTH :]
        wid = lax.axis_index("s") * NC + lax.axis_index("c")
        base = wid * CHUNK
        pltpu.sync_copy(idx_hbm.at[:, pl.ds(base, CHUNK)], idx_v)

        def start_gather(c, b):
            pltpu.async_copy(table_hbm.at[idx_v.at[c]], bufs[b], gsems[b])

        def wait_gather(c, b):
            pltpu.make_async_copy(table_hbm.at[idx_v.at[c]], bufs[b], gsems[b]).wait()

        def out_slice(c):
            return out_hbm.at[pl.ds(c * n_seq + base, CHUNK)]

        def start_out(c, b):
            pltpu.async_copy(bufs[b], out_slice(c), osems[b])

        def wait_out(c, b):
            pltpu.make_async_copy(bufs[b], out_slice(c), osems[b]).wait()

        for b in range(DEPTH):
            start_gather(b, b)

        def body(i, _):
            for b in range(DEPTH):
                c = i * DEPTH + b
                wait_gather(c, b)
                start_out(c, b)

                @pl.when(c + DEPTH < n_chunks)
                def _():
                    wait_out(c, b)
                    start_gather(c + DEPTH, b)

            return 0

        lax.fori_loop(0, n_chunks // DEPTH, body, 0)

        for t in range(tail):
            c = (n_chunks // DEPTH) * DEPTH + t
            wait_gather(c, t)
            start_out(c, t)

        for t in range(DEPTH):
            c = n_chunks - DEPTH + t
            wait_out(c, c % DEPTH)

    return gather


@jax.jit
def kernel(word_sequences, table):
    n_seq, seq_len = word_sequences.shape
    idx = word_sequences.astype(jnp.int32).T
    out = _make_gather(n_seq, seq_len)(table, idx)
    return out.reshape(seq_len, n_seq, EMB).transpose(1, 0, 2)


# speedup vs baseline: 14.1740x; 1.0026x over previous
import functools

import jax
import jax.numpy as jnp
from jax import lax
from jax.experimental import pallas as pl
from jax.experimental.pallas import tpu as pltpu
from jax.experimental.pallas import tpu_sc as plsc

EMB = 128
NC = 2
NS = 16
NW = NC * NS
CHUNK = 128
DEPTH = 7


def _make_gather(n_seq: int, seq_len: int):
    assert n_seq % (NW * CHUNK) == 0 and n_seq // NW == CHUNK
    n_rows = n_seq * seq_len
    n_chunks = seq_len
    assert n_chunks >= 2 * DEPTH
    tail = n_chunks % DEPTH
    mesh = plsc.VectorSubcoreMesh(core_axis_name="c", subcore_axis_name="s")

    @functools.partial(
        pl.kernel,
        mesh=mesh,
        out_type=jax.ShapeDtypeStruct((n_rows, EMB), jnp.float32),
        scratch_types=[
            pltpu.VMEM((n_chunks, CHUNK), jnp.int32),
        ]
        + [pltpu.VMEM((CHUNK, EMB), jnp.float32) for _ in range(DEPTH)]
        + [pltpu.SemaphoreType.DMA for _ in range(2 * DEPTH)],
    )
    def gather(table_hbm, idx_hbm, out_hbm, idx_v, *rest):
        bufs = rest[:DEPTH]
        gsems = rest[DEPTH : 2 * DEPTH]
        osems = rest[2 * DEPTH :]
        wid = lax.axis_index("s") * NC + lax.axis_index("c")
        base = wid * CHUNK
        pltpu.sync_copy(idx_hbm.at[:, pl.ds(base, CHUNK)], idx_v)

        def start_gather(c, b):
            pltpu.async_copy(table_hbm.at[idx_v.at[c]], bufs[b], gsems[b])

        def wait_gather(c, b):
            pltpu.make_async_copy(table_hbm.at[idx_v.at[c]], bufs[b], gsems[b]).wait()

        def out_slice(c):
            return out_hbm.at[pl.ds(c * n_seq + base, CHUNK)]

        def start_out(c, b):
            pltpu.async_copy(bufs[b], out_slice(c), osems[b])

        def wait_out(c, b):
            pltpu.make_async_copy(bufs[b], out_slice(c), osems[b]).wait()

        for b in range(DEPTH):
            start_gather(b, b)

        def body(i, _):
            for b in range(DEPTH):
                c = i * DEPTH + b
                wait_gather(c, b)
                start_out(c, b)

                @pl.when(c + DEPTH < n_chunks)
                def _():
                    wait_out(c, b)
                    start_gather(c + DEPTH, b)

            return 0

        lax.fori_loop(0, n_chunks // DEPTH, body, 0)

        for t in range(tail):
            c = (n_chunks // DEPTH) * DEPTH + t
            wait_gather(c, t)
            start_out(c, t)

        for t in range(DEPTH):
            c = n_chunks - DEPTH + t
            wait_out(c, c % DEPTH)

    return gather


@jax.jit
def kernel(word_sequences, table):
    n_seq, seq_len = word_sequences.shape
    idx = word_sequences.astype(jnp.int32).T
    out = _make_gather(n_seq, seq_len)(table, idx)
    return out.reshape(seq_len, n_seq, EMB).transpose(1, 0, 2)
